# Initial kernel scaffold; baseline (speedup 1.0000x reference)
import jax
import jax.numpy as jnp
from jax.experimental import pallas as pl


def kernel(x, edge_index, edge_weights, batch, W0, b0, g0, be0, Wl0, bl0, theta1, W1, b1, g1, be1, Wl1, bl1, theta2, W2, b2, g2, be2, Wl2, bl2):
    raise NotImplementedError("write your pallas kernel here")


# speedup vs baseline: 3.7265x; 3.7265x over previous
import functools

import jax
import jax.numpy as jnp
from jax import lax
from jax.experimental import pallas as pl
from jax.experimental.pallas import tpu as pltpu
from jax.experimental.pallas import tpu_sc as plsc

N = 10000
E = 320000
F_IN = 128
H = 64
C = 10
NG = 64
CP = 16

NC = 2
NS = 16
NW = NC * NS
K = 128
CH = -(-E // (NW * K))
EPAD = NW * CH * K
NP = 10240
TRASH = N


def _make_sc_segsum(nb):
    acc_rows = nb * NP
    rps = acc_rows // NS
    nz = rps // K

    @functools.partial(
        pl.kernel,
        out_type=jax.ShapeDtypeStruct((NC * acc_rows, H), jnp.float32),
        mesh=plsc.VectorSubcoreMesh(core_axis_name="c", subcore_axis_name="s"),
        compiler_params=pltpu.CompilerParams(use_tc_tiling_on_sc=False),
        scratch_types=[
            pltpu.VMEM((K,), jnp.int32),
            pltpu.VMEM((K,), jnp.int32),
            pltpu.VMEM((K, H), jnp.float32),
            pltpu.VMEM((K, H), jnp.float32),
            pltpu.VMEM_SHARED((acc_rows, H), jnp.float32),
            pltpu.SemaphoreType.DMA,
        ],
    )
    def sc_fn(h_hbm, src_hbm, idx_hbm, zblk_hbm, out_hbm,
              srcbuf, idxbuf, rows, zbuf, acc, sem):
        c = lax.axis_index("c")
        s = lax.axis_index("s")
        wid = s * NC + c

        pltpu.sync_copy(zblk_hbm, zbuf)

        def zbody(k, _):
            pltpu.sync_copy(zbuf, acc.at[pl.ds(s * rps + k * K, K)])
            return 0
        lax.fori_loop(0, nz, zbody, 0)
        plsc.subcore_barrier()

        base = wid * CH * K

        def ebody(j, _):
            off = base + j * K
            pltpu.sync_copy(src_hbm.at[pl.ds(off, K)], srcbuf)
            pltpu.sync_copy(idx_hbm.at[pl.ds(off, K)], idxbuf)
            pltpu.async_copy(h_hbm.at[srcbuf], rows, sem).wait()
            pltpu.sync_copy(rows, acc.at[idxbuf], add=True)
            return 0
        lax.fori_loop(0, CH, ebody, 0)
        plsc.subcore_barrier()

        def xbody(k, _):
            r = s * rps + k * K
            pltpu.sync_copy(acc.at[pl.ds(r, K)], zbuf)
            pltpu.sync_copy(zbuf, out_hbm.at[pl.ds(c * acc_rows + r, K)])
            return 0
        lax.fori_loop(0, nz, xbody, 0)

    return sc_fn


_sc_l1 = _make_sc_segsum(1)
_sc_l2 = _make_sc_segsum(2)


def _bn_relu(z, g, be):
    m = jnp.mean(z, axis=0, keepdims=True)
    v = jnp.mean((z - m) ** 2, axis=0, keepdims=True)
    return jnp.maximum(g * (z - m) * lax.rsqrt(v + 1e-5) + be, 0.0)


def _segmax(p, bat, pool_ref):
    def gbody(g, _):
        mm = jnp.max(jnp.where(bat == g, p, -jnp.inf), axis=0, keepdims=True)
        pool_ref[pl.ds(g, 1), :] = mm
        return 0
    lax.fori_loop(0, NG, gbody, 0)


def _stage0_body(x_ref, w_ref, b_ref, g_ref, be_ref, wl_ref, bl_ref, bat_ref,
                 h_ref, pool_ref):
    z = jnp.dot(x_ref[...], w_ref[...], preferred_element_type=jnp.float32)
    h = _bn_relu(z + b_ref[...], g_ref[...], be_ref[...])
    h_ref[...] = h
    p = jnp.dot(h, wl_ref[...], preferred_element_type=jnp.float32) + bl_ref[...]
    _segmax(p, bat_ref[...], pool_ref)


def _make_stageL_body(nb):
    def body(h_ref, acc_ref, th_ref, w_ref, b_ref, g_ref, be_ref, wl_ref,
             bl_ref, bat_ref, pin_ref, ho_ref, pool_ref):
        agg = jnp.zeros((N, H), jnp.float32)
        for cc in range(NC):
            for d in range(nb):
                agg = agg + th_ref[d] * acc_ref[pl.ds(cc * nb * NP + d * NP, N), :]
        z = jnp.dot(h_ref[...] + agg, w_ref[...],
                    preferred_element_type=jnp.float32)
        h = _bn_relu(z + b_ref[...], g_ref[...], be_ref[...])
        ho_ref[...] = h
        p = jnp.dot(h, wl_ref[...], preferred_element_type=jnp.float32) + bl_ref[...]
        _segmax(p, bat_ref[...], pool_ref)
        pool_ref[...] = pool_ref[...] + pin_ref[...]
    return body


def _tc_stage0(x, w, b, g, be, wl, bl, bat):
    return pl.pallas_call(
        _stage0_body,
        out_shape=(jax.ShapeDtypeStruct((N, H), jnp.float32),
                   jax.ShapeDtypeStruct((NG, CP), jnp.float32)),
    )(x, w, b, g, be, wl, bl, bat)


def _tc_stageL(nb, h, acc, th, w, b, g, be, wl, bl, bat, pin):
    nin = 11
    return pl.pallas_call(
        _make_stageL_body(nb),
        in_specs=[pl.BlockSpec(memory_space=pltpu.SMEM) if i == 2 else
                  pl.BlockSpec(memory_space=pltpu.VMEM) for i in range(nin)],
        out_shape=(jax.ShapeDtypeStruct((N, H), jnp.float32),
                   jax.ShapeDtypeStruct((NG, CP), jnp.float32)),
    )(h, acc, th, w, b, g, be, wl, bl, bat, pin)


def kernel(x, edge_index, edge_weights, batch,
           W0, b0, g0, be0, Wl0, bl0,
           theta1, W1, b1, g1, be1, Wl1, bl1,
           theta2, W2, b2, g2, be2, Wl2, bl2):
    src = edge_index[0]
    dst = edge_index[1]
    ew = edge_weights

    pad = jnp.zeros((EPAD - E,), jnp.int32)
    srcp = jnp.concatenate([src, pad])
    tpad = jnp.full((EPAD - E,), TRASH, jnp.int32)
    idx1 = jnp.concatenate([jnp.where(ew <= 0, dst, TRASH), tpad])
    idx2 = jnp.concatenate([jnp.where(ew <= 1, ew * NP + dst, TRASH), tpad])
    zblk = jnp.zeros((K, H), jnp.float32)

    def row(v):
        return v.reshape(1, -1)
    def padwl(wl):
        return jnp.pad(wl, ((0, 0), (0, CP - C)))
    def padbl(bl):
        return jnp.pad(bl, (0, CP - C)).reshape(1, CP)
    bat = batch.reshape(N, 1)

    h0, p0 = _tc_stage0(x, W0, row(b0), row(g0), row(be0),
                        padwl(Wl0), padbl(bl0), bat)
    acc1 = _sc_l1(h0, srcp, idx1, zblk)
    h1, p1 = _tc_stageL(1, h0, acc1, theta1, W1, row(b1), row(g1), row(be1),
                        padwl(Wl1), padbl(bl1), bat, p0)
    acc2 = _sc_l2(h1, srcp, idx2, zblk)
    _, p2 = _tc_stageL(2, h1, acc2, theta2, W2, row(b2), row(g2), row(be2),
                       padwl(Wl2), padbl(bl2), bat, p1)
    return p2[:, :C]
